# Initial kernel scaffold; baseline (speedup 1.0000x reference)
import jax
import jax.numpy as jnp
from jax.experimental import pallas as pl


def kernel(data, label, cluster, weight, profit):
    raise NotImplementedError("write your pallas kernel here")


# speedup vs baseline: 134.9607x; 134.9607x over previous
import jax
import jax.numpy as jnp
from jax import lax
from jax.experimental import pallas as pl
from jax.experimental.pallas import tpu as pltpu
from jax.experimental.pallas import tpu_sc as plsc

NCLUSTER = 256
NDIM = 128
NPOINTS = 100000
VALIDSIZE = 2000

TN = 512
NPAD = ((NPOINTS + TN - 1) // TN) * TN
NBLK = NPAD // TN

L = 16
NB = 128
NQ = 1024
CAP = 4096
RW = 8
W = 2048
NW = NPAD // W
KV = W // L


def _dist_kernel(dataT_ref, w_ref, wc2_ref, ksum_ref, thd_ref, lab_ref,
                 dist_ref, cnt_ref, slab_ref):
    j = pl.program_id(0)
    dT = dataT_ref[...]
    d2T = dT * dT
    dist = jnp.dot(w_ref[...], d2T, preferred_element_type=jnp.float32)
    dist = dist + jnp.dot(wc2_ref[...], dT, preferred_element_type=jnp.float32)
    dist = dist + ksum_ref[...]
    col = lax.broadcasted_iota(jnp.int32, (1, TN), 1) + j * TN
    dist = jnp.where(col < NPOINTS, dist, 1e30)
    dist_ref[...] = dist
    mask = (dist <= thd_ref[...]).astype(jnp.float32)
    cnt_p = jnp.sum(mask, axis=1, keepdims=True)
    slab_p = jnp.sum(mask * lab_ref[...], axis=1, keepdims=True)

    @pl.when(j == 0)
    def _():
        cnt_ref[...] = jnp.zeros_like(cnt_ref)
        slab_ref[...] = jnp.zeros_like(slab_ref)

    cnt_ref[...] += cnt_p
    slab_ref[...] += slab_p


def _dist_stage(dataT, labp, weight, wc2, ksum, thd):
    return pl.pallas_call(
        _dist_kernel,
        grid=(NBLK,),
        in_specs=[
            pl.BlockSpec((NDIM, TN), lambda j: (0, j)),
            pl.BlockSpec((NCLUSTER, NDIM), lambda j: (0, 0)),
            pl.BlockSpec((NCLUSTER, NDIM), lambda j: (0, 0)),
            pl.BlockSpec((NCLUSTER, 1), lambda j: (0, 0)),
            pl.BlockSpec((NCLUSTER, 1), lambda j: (0, 0)),
            pl.BlockSpec((1, TN), lambda j: (0, j)),
        ],
        out_specs=[
            pl.BlockSpec((NCLUSTER, TN), lambda j: (0, j)),
            pl.BlockSpec((NCLUSTER, 1), lambda j: (0, 0)),
            pl.BlockSpec((NCLUSTER, 1), lambda j: (0, 0)),
        ],
        out_shape=[
            jax.ShapeDtypeStruct((NCLUSTER, NPAD), jnp.float32),
            jax.ShapeDtypeStruct((NCLUSTER, 1), jnp.float32),
            jax.ShapeDtypeStruct((NCLUSTER, 1), jnp.float32),
        ],
    )(dataT, weight, wc2, ksum, thd, labp)


def _select_body(dist_hbm, thd_hbm, out_hbm, buf, thd_b, hist1, cand,
                 qhist, dst, outb):
    wid = lax.axis_index("s") * 2 + lax.axis_index("c")
    c8 = pl.multiple_of(wid * RW, 8)
    lane = lax.iota(jnp.int32, L)
    ones_i = jnp.ones((L,), jnp.int32)
    z16i = jnp.zeros((L,), jnp.int32)
    big = jnp.full((L,), 1e30, jnp.float32)

    pltpu.sync_copy(thd_hbm.at[pl.ds(c8 * L, RW * L)], thd_b)

    def zh(i, _):
        hist1[pl.ds(i * L, L)] = z16i
        return ()
    lax.fori_loop(0, RW * NB * L // L, zh, ())

    def p1(w, _):
        off = pl.multiple_of(w * W, 128)
        pltpu.sync_copy(dist_hbm.at[pl.ds(c8, RW), pl.ds(off, W)], buf)

        def p1r(r, _):
            inv_w = plsc.load_gather(thd_b, [jnp.full((L,), r * L, jnp.int32)])
            inv_w = jnp.float32(NB) / inv_w

            def p1k(k, _):
                x = buf[r, pl.ds(k * L, L)]
                b = jnp.clip((x * inv_w).astype(jnp.int32), 0, NB - 1)
                plsc.addupdate_scatter(
                    hist1, [r * (NB * L) + lane * NB + b], ones_i)
                return ()
            lax.fori_loop(0, KV, p1k, ())
            return ()
        lax.fori_loop(0, RW, p1r, ())
        return ()
    lax.fori_loop(0, NW, p1, ())

    tc_splat = []
    for r in range(RW):
        tv = thd_b[pl.ds(r * L, L)]
        width = tv / jnp.float32(NB)
        base = jnp.int32(0)
        b_star = jnp.int32(0)
        for g in range(NB // L):
            tot = z16i
            for l in range(L):
                tot = tot + hist1[pl.ds(r * (NB * L) + l * NB + g * L, L)]
            cum = plsc.cumsum(tot) + base
            b_star = b_star + jnp.sum((cum < VALIDSIZE).astype(jnp.int32))
            base = base + jnp.sum(tot)
        t_r = (b_star.astype(jnp.float32) + 2.0) * width
        tc_splat.append(t_r)

    def zc(i, _):
        cand[pl.ds(i * L, L)] = big
        return ()
    lax.fori_loop(0, RW * CAP // L, zc, ())

    def p2(w, mofs):
        off = pl.multiple_of(w * W, 128)
        pltpu.sync_copy(dist_hbm.at[pl.ds(c8, RW), pl.ds(off, W)], buf)
        new = []
        for r in range(RW):
            tcr = tc_splat[r]

            def p2k(k, m_off, r=r, tcr=tcr):
                x = buf[r, pl.ds(k * L, L)]
                msk = x <= tcr
                plsc.store_compressed(cand.at[pl.ds(r * CAP + m_off, L)],
                                      x, mask=msk)
                m_off = m_off + jnp.sum(msk.astype(jnp.int32))
                return jnp.minimum(m_off, CAP - L)
            new.append(lax.fori_loop(0, KV, p2k, mofs[r]))
        return tuple(new)

    mofs = lax.fori_loop(0, NW, p2, (jnp.int32(0),) * RW)

    for r in range(RW):
        tv = thd_b[pl.ds(r * L, L)]
        tcr = tc_splat[r]

        def mm(i, carry, r=r, tcr=tcr):
            cmin, cmax = carry
            v = cand[pl.ds(r * CAP + i * L, L)]
            sel = v <= tcr
            cmin = jnp.minimum(cmin, jnp.where(sel, v, 1e30))
            cmax = jnp.maximum(cmax, jnp.where(sel, v, -1e30))
            return cmin, cmax
        cminv, cmaxv = lax.fori_loop(0, CAP // L, mm, (big, -big))
        cmin = jnp.min(cminv)
        cmax = jnp.max(cmaxv)
        span = jnp.broadcast_to(cmax - cmin, (L,))
        qs = jnp.full((L,), NQ, jnp.float32) / jnp.maximum(span, 1e-20)

        def zq(i, _):
            qhist[pl.ds(i * L, L)] = z16i
            return ()
        lax.fori_loop(0, NQ * L // L, zq, ())

        def quant(v):
            qf = jnp.minimum((v - cmin) * qs, jnp.float32(NQ - 1))
            qf = jnp.maximum(qf, jnp.float32(0.0))
            return qf.astype(jnp.int32)

        def qh(i, _, r=r):
            v = cand[pl.ds(r * CAP + i * L, L)]
            plsc.addupdate_scatter(qhist, [lane * NQ + quant(v)], ones_i)
            return ()
        lax.fori_loop(0, CAP // L, qh, ())

        def sc(g, base):
            hl = [qhist[pl.ds(l * NQ + g * L, L)] for l in range(L)]
            tot = z16i
            for l in range(L):
                tot = tot + hl[l]
            excl = plsc.cumsum(tot) - tot + base
            run = z16i
            for l in range(L):
                qhist[pl.ds(l * NQ + g * L, L)] = excl + run
                run = run + hl[l]
            return base + jnp.sum(tot)
        lax.fori_loop(0, NQ // L, sc, jnp.int32(0))

        def st(i, _, r=r):
            v = cand[pl.ds(r * CAP + i * L, L)]
            qi = lane * NQ + quant(v)
            pos = plsc.load_gather(qhist, [qi])
            plsc.store_scatter(dst, [pos], v)
            plsc.addupdate_scatter(qhist, [qi], ones_i)
            return ()
        lax.fori_loop(0, CAP // L, st, ())

        def em(i, _, r=r, tv=tv):
            v = dst[pl.ds(i * L, L)]
            outb[r, pl.ds(i * L, L)] = jnp.where(v <= tv, v, 0.0)
            return ()
        lax.fori_loop(0, VALIDSIZE // L, em, ())

    pltpu.sync_copy(outb, out_hbm.at[pl.ds(c8, RW), :])


def _select_stage(dist, thd16):
    mesh = plsc.VectorSubcoreMesh(core_axis_name="c", subcore_axis_name="s")
    return pl.kernel(
        _select_body,
        out_type=jax.ShapeDtypeStruct((NCLUSTER, VALIDSIZE), jnp.float32),
        mesh=mesh,
        scratch_types=[
            pltpu.VMEM((RW, W), jnp.float32),
            pltpu.VMEM((RW * L,), jnp.float32),
            pltpu.VMEM((RW * NB * L,), jnp.int32),
            pltpu.VMEM((RW * CAP,), jnp.float32),
            pltpu.VMEM((NQ * L,), jnp.int32),
            pltpu.VMEM((CAP,), jnp.float32),
            pltpu.VMEM((RW, VALIDSIZE), jnp.float32),
        ],
        compiler_params=pltpu.CompilerParams(needs_layout_passes=False),
    )(dist, thd16)


def kernel(data, label, cluster, weight, profit):
    dataT = jnp.pad(data, ((0, NPAD - NPOINTS), (0, 0))).T
    labp = jnp.pad(label, (0, NPAD - NPOINTS)).reshape(1, NPAD)
    wc = weight * cluster
    wc2 = -2.0 * wc
    ksum = jnp.sum(wc * cluster, axis=1, keepdims=True)
    thd = profit[:, 3:4]
    dist, cnt, slab = _dist_stage(dataT, labp, weight, wc2, ksum, thd)
    thd16 = jnp.broadcast_to(thd, (NCLUSTER, L)).reshape(-1)
    sel_vals = _select_stage(dist, thd16)
    cnt1 = cnt[:, 0]
    prof = slab[:, 0] / jnp.maximum(cnt1, 1.0)
    stat = jnp.stack([prof, cnt1], axis=1)
    return sel_vals, stat


# speedup vs baseline: 151.1796x; 1.1202x over previous
import jax
import jax.numpy as jnp
from jax import lax
from jax.experimental import pallas as pl
from jax.experimental.pallas import tpu as pltpu
from jax.experimental.pallas import tpu_sc as plsc

NCLUSTER = 256
NDIM = 128
NPOINTS = 100000
VALIDSIZE = 2000

TN = 512
NPAD = ((NPOINTS + TN - 1) // TN) * TN
NBLK = NPAD // TN

L = 16
NB = 128
NQ = 1024
CAP = 4096
RW = 8
W = 2048
NW = NPAD // W
KV = W // L
U1 = 4
U2 = 4


def _dist_kernel(dataT_ref, w_ref, wc2_ref, ksum_ref, thd_ref, lab_ref,
                 dist_ref, cnt_ref, slab_ref):
    j = pl.program_id(0)
    dT = dataT_ref[...]
    d2T = dT * dT
    dist = jnp.dot(w_ref[...], d2T, preferred_element_type=jnp.float32)
    dist = dist + jnp.dot(wc2_ref[...], dT, preferred_element_type=jnp.float32)
    dist = dist + ksum_ref[...]
    col = lax.broadcasted_iota(jnp.int32, (1, TN), 1) + j * TN
    dist = jnp.where(col < NPOINTS, dist, 1e30)
    dist_ref[...] = dist
    mask = (dist <= thd_ref[...]).astype(jnp.float32)
    cnt_p = jnp.sum(mask, axis=1, keepdims=True)
    slab_p = jnp.sum(mask * lab_ref[...], axis=1, keepdims=True)

    @pl.when(j == 0)
    def _():
        cnt_ref[...] = jnp.zeros_like(cnt_ref)
        slab_ref[...] = jnp.zeros_like(slab_ref)

    cnt_ref[...] += cnt_p
    slab_ref[...] += slab_p


def _dist_stage(dataT, labp, weight, wc2, ksum, thd):
    return pl.pallas_call(
        _dist_kernel,
        grid=(NBLK,),
        in_specs=[
            pl.BlockSpec((NDIM, TN), lambda j: (0, j)),
            pl.BlockSpec((NCLUSTER, NDIM), lambda j: (0, 0)),
            pl.BlockSpec((NCLUSTER, NDIM), lambda j: (0, 0)),
            pl.BlockSpec((NCLUSTER, 1), lambda j: (0, 0)),
            pl.BlockSpec((NCLUSTER, 1), lambda j: (0, 0)),
            pl.BlockSpec((1, TN), lambda j: (0, j)),
        ],
        out_specs=[
            pl.BlockSpec((NCLUSTER, TN), lambda j: (0, j)),
            pl.BlockSpec((NCLUSTER, 1), lambda j: (0, 0)),
            pl.BlockSpec((NCLUSTER, 1), lambda j: (0, 0)),
        ],
        out_shape=[
            jax.ShapeDtypeStruct((NCLUSTER, NPAD), jnp.float32),
            jax.ShapeDtypeStruct((NCLUSTER, 1), jnp.float32),
            jax.ShapeDtypeStruct((NCLUSTER, 1), jnp.float32),
        ],
    )(dataT, weight, wc2, ksum, thd, labp)


def _hist_rows(buf, hist1, inv_ws, lane, ones_i):
    for r in range(RW):
        inv_w = inv_ws[r]
        base_r = r * (NB * L)

        def hk(k, _, r=r, inv_w=inv_w, base_r=base_r):
            for u in range(U1):
                x = buf[r, pl.ds((k * U1 + u) * L, L)]
                b = jnp.clip((x * inv_w).astype(jnp.int32), 0, NB - 1)
                plsc.addupdate_scatter(hist1, [base_r + lane * NB + b], ones_i)
            return ()
        lax.fori_loop(0, KV // U1, hk, ())


def _compact_rows(buf, cand, tc_splat, mofs):
    new = []
    for r in range(RW):
        tcr = tc_splat[r]

        def ck(k, m_off, r=r, tcr=tcr):
            for u in range(U2):
                x = buf[r, pl.ds((k * U2 + u) * L, L)]
                msk = x <= tcr
                plsc.store_compressed(cand.at[pl.ds(r * CAP + m_off, L)],
                                      x, mask=msk)
                pc = plsc.all_reduce_population_count(msk)
                m_off = jnp.minimum(m_off + pc[0], CAP - L)
            return m_off
        new.append(lax.fori_loop(0, KV // U2, ck, mofs[r]))
    return tuple(new)


def _select_body(dist_hbm, thd_hbm, out_hbm, bufa, bufb, thd_b, hist1, cand,
                 qhist, dst, outb, sema, semb):
    wid = lax.axis_index("s") * 2 + lax.axis_index("c")
    c8 = pl.multiple_of(wid * RW, 8)
    lane = lax.iota(jnp.int32, L)
    ones_i = jnp.ones((L,), jnp.int32)
    z16i = jnp.zeros((L,), jnp.int32)
    big = jnp.full((L,), 1e30, jnp.float32)

    pltpu.sync_copy(thd_hbm.at[pl.ds(c8 * L, RW * L)], thd_b)
    inv_ws, tvs = [], []
    for r in range(RW):
        tv = thd_b[pl.ds(r * L, L)]
        tvs.append(tv)
        inv_ws.append(jnp.float32(NB) / tv)

    def zh(i, _):
        for u in range(4):
            hist1[pl.ds((i * 4 + u) * L, L)] = z16i
        return ()
    lax.fori_loop(0, RW * NB // 4, zh, ())

    def start(w, buf, sem):
        off = pl.multiple_of(w * W, 128)
        return pltpu.async_copy(
            dist_hbm.at[pl.ds(c8, RW), pl.ds(off, W)], buf, sem)

    def wait(buf, sem):
        pltpu.make_async_copy(
            dist_hbm.at[pl.ds(c8, RW), pl.ds(0, W)], buf, sem).wait()

    start(0, bufa, sema)

    def p1(wp, _):
        start(2 * wp + 1, bufb, semb)
        wait(bufa, sema)
        _hist_rows(bufa, hist1, inv_ws, lane, ones_i)
        start(2 * wp + 2, bufa, sema)
        wait(bufb, semb)
        _hist_rows(bufb, hist1, inv_ws, lane, ones_i)
        return ()
    lax.fori_loop(0, (NW - 1) // 2, p1, ())
    wait(bufa, sema)
    _hist_rows(bufa, hist1, inv_ws, lane, ones_i)

    tc_splat = []
    for r in range(RW):
        width = tvs[r] / jnp.float32(NB)

        def sg(g, carry, r=r):
            base, b_star = carry
            tot = z16i
            for l in range(L):
                tot = tot + hist1[pl.ds(r * (NB * L) + l * NB + g * L, L)]
            cum = plsc.cumsum(tot) + base
            b_star = b_star + jnp.sum((cum < VALIDSIZE).astype(jnp.int32))
            return base + jnp.sum(tot), b_star
        _, b_star = lax.fori_loop(0, NB // L, sg, (jnp.int32(0), jnp.int32(0)))
        tc_splat.append((b_star.astype(jnp.float32) + 2.0) * width)

    def zc(i, _):
        for u in range(4):
            cand[pl.ds((i * 4 + u) * L, L)] = big
        return ()
    lax.fori_loop(0, RW * CAP // L // 4, zc, ())

    start(0, bufa, sema)

    def p2(wp, mofs):
        start(2 * wp + 1, bufb, semb)
        wait(bufa, sema)
        mofs = _compact_rows(bufa, cand, tc_splat, mofs)
        start(2 * wp + 2, bufa, sema)
        wait(bufb, semb)
        return _compact_rows(bufb, cand, tc_splat, mofs)
    mofs = lax.fori_loop(0, (NW - 1) // 2, p2, (jnp.int32(0),) * RW)
    wait(bufa, sema)
    mofs = _compact_rows(bufa, cand, tc_splat, mofs)

    for r in range(RW):
        tv = tvs[r]
        tcr = tc_splat[r]

        def mm(i, carry, r=r, tcr=tcr):
            cmin, cmax = carry
            for u in range(4):
                v = cand[pl.ds(r * CAP + (i * 4 + u) * L, L)]
                sel = v <= tcr
                cmin = jnp.minimum(cmin, jnp.where(sel, v, 1e30))
                cmax = jnp.maximum(cmax, jnp.where(sel, v, -1e30))
            return cmin, cmax
        cminv, cmaxv = lax.fori_loop(0, CAP // L // 4, mm, (big, -big))
        cmin = jnp.min(cminv)
        cmax = jnp.max(cmaxv)
        span = jnp.broadcast_to(cmax - cmin, (L,))
        qs = jnp.full((L,), NQ, jnp.float32) / jnp.maximum(span, 1e-20)

        def zq(i, _):
            for u in range(4):
                qhist[pl.ds((i * 4 + u) * L, L)] = z16i
            return ()
        lax.fori_loop(0, NQ * L // L // 4, zq, ())

        def quant(v):
            qf = jnp.minimum((v - cmin) * qs, jnp.float32(NQ - 1))
            qf = jnp.maximum(qf, jnp.float32(0.0))
            return qf.astype(jnp.int32)

        def qh(i, _, r=r):
            for u in range(2):
                v = cand[pl.ds(r * CAP + (i * 2 + u) * L, L)]
                plsc.addupdate_scatter(qhist, [lane * NQ + quant(v)], ones_i)
            return ()
        lax.fori_loop(0, CAP // L // 2, qh, ())

        def sc(g, base):
            hl = [qhist[pl.ds(l * NQ + g * L, L)] for l in range(L)]
            tot = z16i
            for l in range(L):
                tot = tot + hl[l]
            excl = plsc.cumsum(tot) - tot + base
            run = z16i
            for l in range(L):
                qhist[pl.ds(l * NQ + g * L, L)] = excl + run
                run = run + hl[l]
            return base + jnp.sum(tot)
        lax.fori_loop(0, NQ // L, sc, jnp.int32(0))

        def st(i, _, r=r):
            v = cand[pl.ds(r * CAP + i * L, L)]
            qi = lane * NQ + quant(v)
            pos = plsc.load_gather(qhist, [qi])
            plsc.store_scatter(dst, [pos], v)
            plsc.addupdate_scatter(qhist, [qi], ones_i)
            return ()
        lax.fori_loop(0, CAP // L, st, ())

        def em(i, _, r=r, tv=tv):
            for u in range(5):
                v = dst[pl.ds((i * 5 + u) * L, L)]
                outb[r, pl.ds((i * 5 + u) * L, L)] = jnp.where(v <= tv, v, 0.0)
            return ()
        lax.fori_loop(0, VALIDSIZE // L // 5, em, ())

    pltpu.sync_copy(outb, out_hbm.at[pl.ds(c8, RW), :])


def _select_stage(dist, thd16):
    mesh = plsc.VectorSubcoreMesh(core_axis_name="c", subcore_axis_name="s")
    return pl.kernel(
        _select_body,
        out_type=jax.ShapeDtypeStruct((NCLUSTER, VALIDSIZE), jnp.float32),
        mesh=mesh,
        scratch_types=[
            pltpu.VMEM((RW, W), jnp.float32),
            pltpu.VMEM((RW, W), jnp.float32),
            pltpu.VMEM((RW * L,), jnp.float32),
            pltpu.VMEM((RW * NB * L,), jnp.int32),
            pltpu.VMEM((RW * CAP,), jnp.float32),
            pltpu.VMEM((NQ * L,), jnp.int32),
            pltpu.VMEM((CAP,), jnp.float32),
            pltpu.VMEM((RW, VALIDSIZE), jnp.float32),
            pltpu.SemaphoreType.DMA,
            pltpu.SemaphoreType.DMA,
        ],
        compiler_params=pltpu.CompilerParams(needs_layout_passes=False),
    )(dist, thd16)


def kernel(data, label, cluster, weight, profit):
    dataT = jnp.pad(data, ((0, NPAD - NPOINTS), (0, 0))).T
    labp = jnp.pad(label, (0, NPAD - NPOINTS)).reshape(1, NPAD)
    wc = weight * cluster
    wc2 = -2.0 * wc
    ksum = jnp.sum(wc * cluster, axis=1, keepdims=True)
    thd = profit[:, 3:4]
    dist, cnt, slab = _dist_stage(dataT, labp, weight, wc2, ksum, thd)
    thd16 = jnp.broadcast_to(thd, (NCLUSTER, L)).reshape(-1)
    sel_vals = _select_stage(dist, thd16)
    cnt1 = cnt[:, 0]
    prof = slab[:, 0] / jnp.maximum(cnt1, 1.0)
    stat = jnp.stack([prof, cnt1], axis=1)
    return sel_vals, stat


# speedup vs baseline: 172.3858x; 1.1403x over previous
import jax
import jax.numpy as jnp
from jax import lax
from jax.experimental import pallas as pl
from jax.experimental.pallas import tpu as pltpu
from jax.experimental.pallas import tpu_sc as plsc

NCLUSTER = 256
NDIM = 128
NPOINTS = 100000
VALIDSIZE = 2000

TN = 512
NPAD = ((NPOINTS + TN - 1) // TN) * TN
NBLK = NPAD // TN

L = 16
NB = 128
NQ = 1024
CAP = 4096
RW = 8
W = 2048
NW = NPAD // W
KV = W // L
U1 = 4
U2 = 4


def _dist_kernel(dataT_ref, w_ref, wc2_ref, ksum_ref, thd_ref, lab_ref,
                 dist_ref, cnt_ref, slab_ref):
    j = pl.program_id(0)
    dT = dataT_ref[...]
    d2T = dT * dT
    dist = jnp.dot(w_ref[...], d2T, preferred_element_type=jnp.float32)
    dist = dist + jnp.dot(wc2_ref[...], dT, preferred_element_type=jnp.float32)
    dist = dist + ksum_ref[...]
    col = lax.broadcasted_iota(jnp.int32, (1, TN), 1) + j * TN
    dist = jnp.where(col < NPOINTS, dist, 1e30)
    dist_ref[...] = dist
    mask = (dist <= thd_ref[...]).astype(jnp.float32)
    cnt_p = jnp.sum(mask, axis=1, keepdims=True)
    slab_p = jnp.sum(mask * lab_ref[...], axis=1, keepdims=True)

    @pl.when(j == 0)
    def _():
        cnt_ref[...] = jnp.zeros_like(cnt_ref)
        slab_ref[...] = jnp.zeros_like(slab_ref)

    cnt_ref[...] += cnt_p
    slab_ref[...] += slab_p


def _dist_stage(dataT, labp, weight, wc2, ksum, thd):
    return pl.pallas_call(
        _dist_kernel,
        grid=(NBLK,),
        in_specs=[
            pl.BlockSpec((NDIM, TN), lambda j: (0, j)),
            pl.BlockSpec((NCLUSTER, NDIM), lambda j: (0, 0)),
            pl.BlockSpec((NCLUSTER, NDIM), lambda j: (0, 0)),
            pl.BlockSpec((NCLUSTER, 1), lambda j: (0, 0)),
            pl.BlockSpec((NCLUSTER, 1), lambda j: (0, 0)),
            pl.BlockSpec((1, TN), lambda j: (0, j)),
        ],
        out_specs=[
            pl.BlockSpec((NCLUSTER, TN), lambda j: (0, j)),
            pl.BlockSpec((NCLUSTER, 1), lambda j: (0, 0)),
            pl.BlockSpec((NCLUSTER, 1), lambda j: (0, 0)),
        ],
        out_shape=[
            jax.ShapeDtypeStruct((NCLUSTER, NPAD), jnp.float32),
            jax.ShapeDtypeStruct((NCLUSTER, 1), jnp.float32),
            jax.ShapeDtypeStruct((NCLUSTER, 1), jnp.float32),
        ],
    )(dataT, weight, wc2, ksum, thd, labp)


def _hist_rows(buf, hist1, inv_ws, lane, ones_i):
    for r in range(RW):
        inv_w = inv_ws[r]
        base_r = r * (NB * L)

        def hk(k, _, r=r, inv_w=inv_w, base_r=base_r):
            for u in range(U1):
                x = buf[r, pl.ds((k * U1 + u) * L, L)]
                b = jnp.clip((x * inv_w).astype(jnp.int32), 0, NB - 1)
                plsc.addupdate_scatter(hist1, [base_r + b * L + lane], ones_i)
            return ()
        lax.fori_loop(0, KV // U1, hk, ())


def _compact_rows(buf, cand, tc_splat, mofs):
    new = []
    for r in range(RW):
        tcr = tc_splat[r]

        def ck(k, m_off, r=r, tcr=tcr):
            for u in range(U2):
                x = buf[r, pl.ds((k * U2 + u) * L, L)]
                msk = x <= tcr
                plsc.store_compressed(cand.at[pl.ds(r * CAP + m_off, L)],
                                      x, mask=msk)
                pc = plsc.all_reduce_population_count(msk)
                m_off = jnp.minimum(m_off + pc[0], CAP - L)
            return m_off
        new.append(lax.fori_loop(0, KV // U2, ck, mofs[r]))
    return tuple(new)


def _select_body(dist_hbm, thd_hbm, out_hbm, bufa, bufb, thd_b, hist1, cand,
                 qhist, dst, outb, sema, semb):
    wid = lax.axis_index("s") * 2 + lax.axis_index("c")
    c8 = pl.multiple_of(wid * RW, 8)
    lane = lax.iota(jnp.int32, L)
    ones_i = jnp.ones((L,), jnp.int32)
    z16i = jnp.zeros((L,), jnp.int32)
    big = jnp.full((L,), 1e30, jnp.float32)

    pltpu.sync_copy(thd_hbm.at[pl.ds(c8 * L, RW * L)], thd_b)
    inv_ws, tvs = [], []
    for r in range(RW):
        tv = thd_b[pl.ds(r * L, L)]
        tvs.append(tv)
        inv_ws.append(jnp.float32(NB) / tv)

    def zh(i, _):
        for u in range(4):
            hist1[pl.ds((i * 4 + u) * L, L)] = z16i
        return ()
    lax.fori_loop(0, RW * NB // 4, zh, ())

    def start(w, buf, sem):
        off = pl.multiple_of(w * W, 128)
        return pltpu.async_copy(
            dist_hbm.at[pl.ds(c8, RW), pl.ds(off, W)], buf, sem)

    def wait(buf, sem):
        pltpu.make_async_copy(
            dist_hbm.at[pl.ds(c8, RW), pl.ds(0, W)], buf, sem).wait()

    start(0, bufa, sema)

    def p1(wp, _):
        start(2 * wp + 1, bufb, semb)
        wait(bufa, sema)
        _hist_rows(bufa, hist1, inv_ws, lane, ones_i)
        start(2 * wp + 2, bufa, sema)
        wait(bufb, semb)
        _hist_rows(bufb, hist1, inv_ws, lane, ones_i)
        return ()
    lax.fori_loop(0, (NW - 1) // 2, p1, ())
    wait(bufa, sema)
    _hist_rows(bufa, hist1, inv_ws, lane, ones_i)

    tc_splat = []
    for r in range(RW):
        width = tvs[r] / jnp.float32(NB)

        def sg(g, carry, r=r):
            base, b_star = carry
            for u in range(4):
                v = hist1[pl.ds(r * (NB * L) + (g * 4 + u) * L, L)]
                tot = jnp.sum(v)
                base = base + tot
                b_star = b_star + (base < VALIDSIZE).astype(jnp.int32)
            return base, b_star
        _, b_star = lax.fori_loop(0, NB // 4, sg, (jnp.int32(0), jnp.int32(0)))
        tc_splat.append((b_star.astype(jnp.float32) + 2.0) * width)

    def zc(i, _):
        for u in range(4):
            cand[pl.ds((i * 4 + u) * L, L)] = big
        return ()
    lax.fori_loop(0, RW * CAP // L // 4, zc, ())

    start(0, bufa, sema)

    def p2(wp, mofs):
        start(2 * wp + 1, bufb, semb)
        wait(bufa, sema)
        mofs = _compact_rows(bufa, cand, tc_splat, mofs)
        start(2 * wp + 2, bufa, sema)
        wait(bufb, semb)
        return _compact_rows(bufb, cand, tc_splat, mofs)
    mofs = lax.fori_loop(0, (NW - 1) // 2, p2, (jnp.int32(0),) * RW)
    wait(bufa, sema)
    mofs = _compact_rows(bufa, cand, tc_splat, mofs)

    for r in range(RW):
        tv = tvs[r]
        tcr = tc_splat[r]

        def mm(i, carry, r=r, tcr=tcr):
            cmin, cmax = carry
            for u in range(4):
                v = cand[pl.ds(r * CAP + (i * 4 + u) * L, L)]
                sel = v <= tcr
                cmin = jnp.minimum(cmin, jnp.where(sel, v, 1e30))
                cmax = jnp.maximum(cmax, jnp.where(sel, v, -1e30))
            return cmin, cmax
        cminv, cmaxv = lax.fori_loop(0, CAP // L // 4, mm, (big, -big))
        cmin = jnp.min(cminv)
        cmax = jnp.max(cmaxv)
        span = jnp.broadcast_to(cmax - cmin, (L,))
        qs = jnp.full((L,), NQ, jnp.float32) / jnp.maximum(span, 1e-20)

        def zq(i, _):
            for u in range(4):
                qhist[pl.ds((i * 4 + u) * L, L)] = z16i
            return ()
        lax.fori_loop(0, NQ * L // L // 4, zq, ())

        def quant(v):
            qf = jnp.minimum((v - cmin) * qs, jnp.float32(NQ - 1))
            qf = jnp.maximum(qf, jnp.float32(0.0))
            return qf.astype(jnp.int32)

        def qh(i, _, r=r):
            for u in range(2):
                v = cand[pl.ds(r * CAP + (i * 2 + u) * L, L)]
                plsc.addupdate_scatter(qhist, [quant(v) * L + lane], ones_i)
            return ()
        lax.fori_loop(0, CAP // L // 2, qh, ())

        def sc(g, base):
            for u in range(2):
                v = qhist[pl.ds((g * 2 + u) * L, L)]
                excl = plsc.cumsum(v) - v + base
                qhist[pl.ds((g * 2 + u) * L, L)] = excl
                base = base + jnp.sum(v)
            return base
        lax.fori_loop(0, NQ // 2, sc, jnp.int32(0))

        def st(i, _, r=r):
            v = cand[pl.ds(r * CAP + i * L, L)]
            qi = quant(v) * L + lane
            pos = plsc.load_gather(qhist, [qi])
            plsc.store_scatter(dst, [pos], v)
            plsc.addupdate_scatter(qhist, [qi], ones_i)
            return ()
        lax.fori_loop(0, CAP // L, st, ())

        def em(i, _, r=r, tv=tv):
            for u in range(5):
                v = dst[pl.ds((i * 5 + u) * L, L)]
                outb[r, pl.ds((i * 5 + u) * L, L)] = jnp.where(v <= tv, v, 0.0)
            return ()
        lax.fori_loop(0, VALIDSIZE // L // 5, em, ())

    pltpu.sync_copy(outb, out_hbm.at[pl.ds(c8, RW), :])


def _select_stage(dist, thd16):
    mesh = plsc.VectorSubcoreMesh(core_axis_name="c", subcore_axis_name="s")
    return pl.kernel(
        _select_body,
        out_type=jax.ShapeDtypeStruct((NCLUSTER, VALIDSIZE), jnp.float32),
        mesh=mesh,
        scratch_types=[
            pltpu.VMEM((RW, W), jnp.float32),
            pltpu.VMEM((RW, W), jnp.float32),
            pltpu.VMEM((RW * L,), jnp.float32),
            pltpu.VMEM((RW * NB * L,), jnp.int32),
            pltpu.VMEM((RW * CAP,), jnp.float32),
            pltpu.VMEM((NQ * L,), jnp.int32),
            pltpu.VMEM((CAP,), jnp.float32),
            pltpu.VMEM((RW, VALIDSIZE), jnp.float32),
            pltpu.SemaphoreType.DMA,
            pltpu.SemaphoreType.DMA,
        ],
        compiler_params=pltpu.CompilerParams(needs_layout_passes=False),
    )(dist, thd16)


def kernel(data, label, cluster, weight, profit):
    dataT = jnp.pad(data, ((0, NPAD - NPOINTS), (0, 0))).T
    labp = jnp.pad(label, (0, NPAD - NPOINTS)).reshape(1, NPAD)
    wc = weight * cluster
    wc2 = -2.0 * wc
    ksum = jnp.sum(wc * cluster, axis=1, keepdims=True)
    thd = profit[:, 3:4]
    dist, cnt, slab = _dist_stage(dataT, labp, weight, wc2, ksum, thd)
    thd16 = jnp.broadcast_to(thd, (NCLUSTER, L)).reshape(-1)
    sel_vals = _select_stage(dist, thd16)
    cnt1 = cnt[:, 0]
    prof = slab[:, 0] / jnp.maximum(cnt1, 1.0)
    stat = jnp.stack([prof, cnt1], axis=1)
    return sel_vals, stat


# speedup vs baseline: 262.1997x; 1.5210x over previous
import jax
import jax.numpy as jnp
from jax import lax
from jax.experimental import pallas as pl
from jax.experimental.pallas import tpu as pltpu
from jax.experimental.pallas import tpu_sc as plsc

NCLUSTER = 256
NDIM = 128
NPOINTS = 100000
VALIDSIZE = 2000

TN = 512
NPAD = ((NPOINTS + TN - 1) // TN) * TN
NBLK = NPAD // TN

L = 16
NB = 128
NQ = 512
NH = 2
CAP = 4096
RW = 8
W = 2048
NW = NPAD // W
KV = W // L
U1 = 4
U2 = 4


def _dist_kernel(dataT_ref, w_ref, wc2_ref, ksum_ref, thd_ref, lab_ref,
                 dist_ref, cnt_ref, slab_ref):
    j = pl.program_id(0)
    dT = dataT_ref[...]
    d2T = dT * dT
    dist = jnp.dot(w_ref[...], d2T, preferred_element_type=jnp.float32)
    dist = dist + jnp.dot(wc2_ref[...], dT, preferred_element_type=jnp.float32)
    dist = dist + ksum_ref[...]
    col = lax.broadcasted_iota(jnp.int32, (1, TN), 1) + j * TN
    dist = jnp.where(col < NPOINTS, dist, 1e30)
    dist_ref[...] = dist
    mask = (dist <= thd_ref[...]).astype(jnp.float32)
    cnt_p = jnp.sum(mask, axis=1, keepdims=True)
    slab_p = jnp.sum(mask * lab_ref[...], axis=1, keepdims=True)

    @pl.when(j == 0)
    def _():
        cnt_ref[...] = jnp.zeros_like(cnt_ref)
        slab_ref[...] = jnp.zeros_like(slab_ref)

    cnt_ref[...] += cnt_p
    slab_ref[...] += slab_p


def _dist_stage(dataT, labp, weight, wc2, ksum, thd):
    return pl.pallas_call(
        _dist_kernel,
        grid=(NBLK,),
        in_specs=[
            pl.BlockSpec((NDIM, TN), lambda j: (0, j)),
            pl.BlockSpec((NCLUSTER, NDIM), lambda j: (0, 0)),
            pl.BlockSpec((NCLUSTER, NDIM), lambda j: (0, 0)),
            pl.BlockSpec((NCLUSTER, 1), lambda j: (0, 0)),
            pl.BlockSpec((NCLUSTER, 1), lambda j: (0, 0)),
            pl.BlockSpec((1, TN), lambda j: (0, j)),
        ],
        out_specs=[
            pl.BlockSpec((NCLUSTER, TN), lambda j: (0, j)),
            pl.BlockSpec((NCLUSTER, 1), lambda j: (0, 0)),
            pl.BlockSpec((NCLUSTER, 1), lambda j: (0, 0)),
        ],
        out_shape=[
            jax.ShapeDtypeStruct((NCLUSTER, NPAD), jnp.float32),
            jax.ShapeDtypeStruct((NCLUSTER, 1), jnp.float32),
            jax.ShapeDtypeStruct((NCLUSTER, 1), jnp.float32),
        ],
    )(dataT, weight, wc2, ksum, thd, labp)


def _hist_rows(buf, hist1, inv_ws, lane, ones_i):
    for r in range(RW):
        inv_w = inv_ws[r]
        base_r = r * (NB * NH * L)

        @plsc.parallel_loop(0, W, step=U1 * L)
        def _(i, r=r, inv_w=inv_w, base_r=base_r):
            for u in range(U1):
                x = buf[r, pl.ds(i + u * L, L)]
                b = jnp.clip((x * inv_w).astype(jnp.int32), 0, NB - 1)
                plsc.addupdate_scatter(
                    hist1, [base_r + b * (NH * L) + (u % NH) * L + lane],
                    ones_i)


def _compact_rows(buf, cand, tc_splat, mofs):
    new = []
    for r in range(RW):
        tcr = tc_splat[r]

        @plsc.parallel_loop(0, W, step=U2 * L, carry=mofs[r])
        def m_fin(i, m_off, r=r, tcr=tcr):
            for u in range(U2):
                x = buf[r, pl.ds(i + u * L, L)]
                msk = x <= tcr
                plsc.store_compressed(cand.at[pl.ds(r * CAP + m_off, L)],
                                      x, mask=msk)
                pc = plsc.all_reduce_population_count(msk)
                m_off = jnp.minimum(m_off + pc[0], CAP - L)
            return m_off
        new.append(m_fin)
    return tuple(new)


def _select_body(dist_hbm, thd_hbm, out_hbm, bufa, bufb, thd_b, hist1, cand,
                 qhist, dst, outb, sema, semb):
    wid = lax.axis_index("s") * 2 + lax.axis_index("c")
    c8 = pl.multiple_of(wid * RW, 8)
    lane = lax.iota(jnp.int32, L)
    ones_i = jnp.ones((L,), jnp.int32)
    z16i = jnp.zeros((L,), jnp.int32)
    big = jnp.full((L,), 1e30, jnp.float32)

    pltpu.sync_copy(thd_hbm.at[pl.ds(c8 * L, RW * L)], thd_b)
    inv_ws, tvs = [], []
    for r in range(RW):
        tv = thd_b[pl.ds(r * L, L)]
        tvs.append(tv)
        inv_ws.append(jnp.float32(NB) / tv)

    @plsc.parallel_loop(0, RW * NB * NH * L, step=4 * L)
    def _(i):
        for u in range(4):
            hist1[pl.ds(i + u * L, L)] = z16i

    def start(w, buf, sem):
        off = pl.multiple_of(w * W, 128)
        return pltpu.async_copy(
            dist_hbm.at[pl.ds(c8, RW), pl.ds(off, W)], buf, sem)

    def wait(buf, sem):
        pltpu.make_async_copy(
            dist_hbm.at[pl.ds(c8, RW), pl.ds(0, W)], buf, sem).wait()

    start(0, bufa, sema)

    def p1(wp, _):
        start(2 * wp + 1, bufb, semb)
        wait(bufa, sema)
        _hist_rows(bufa, hist1, inv_ws, lane, ones_i)
        start(2 * wp + 2, bufa, sema)
        wait(bufb, semb)
        _hist_rows(bufb, hist1, inv_ws, lane, ones_i)
        return ()
    lax.fori_loop(0, (NW - 1) // 2, p1, ())
    wait(bufa, sema)
    _hist_rows(bufa, hist1, inv_ws, lane, ones_i)

    tc_splat = []
    for r in range(RW):
        width = tvs[r] / jnp.float32(NB)

        def sg(g, carry, r=r):
            base, b_star = carry
            for u in range(4):
                cell = r * (NB * NH * L) + (g * 4 + u) * (NH * L)
                v = hist1[pl.ds(cell, L)]
                for h in range(1, NH):
                    v = v + hist1[pl.ds(cell + h * L, L)]
                base = base + jnp.sum(v)
                b_star = b_star + (base < VALIDSIZE).astype(jnp.int32)
            return base, b_star
        _, b_star = lax.fori_loop(0, NB // 4, sg, (jnp.int32(0), jnp.int32(0)))
        tc_splat.append((b_star.astype(jnp.float32) + 2.0) * width)

    @plsc.parallel_loop(0, RW * CAP, step=4 * L)
    def _(i):
        for u in range(4):
            cand[pl.ds(i + u * L, L)] = big

    start(0, bufa, sema)

    def p2(wp, mofs):
        start(2 * wp + 1, bufb, semb)
        wait(bufa, sema)
        mofs = _compact_rows(bufa, cand, tc_splat, mofs)
        start(2 * wp + 2, bufa, sema)
        wait(bufb, semb)
        return _compact_rows(bufb, cand, tc_splat, mofs)
    mofs = lax.fori_loop(0, (NW - 1) // 2, p2, (jnp.int32(0),) * RW)
    wait(bufa, sema)
    mofs = _compact_rows(bufa, cand, tc_splat, mofs)

    for r in range(RW):
        tv = tvs[r]
        tcr = tc_splat[r]

        @plsc.parallel_loop(0, CAP, step=4 * L, carry=(big, -big))
        def mm_fin(i, carry, r=r, tcr=tcr):
            cmin, cmax = carry
            for u in range(4):
                v = cand[pl.ds(r * CAP + i + u * L, L)]
                sel = v <= tcr
                cmin = jnp.minimum(cmin, jnp.where(sel, v, 1e30))
                cmax = jnp.maximum(cmax, jnp.where(sel, v, -1e30))
            return cmin, cmax
        cminv, cmaxv = mm_fin
        cmin = jnp.min(cminv)
        cmax = jnp.max(cmaxv)
        span = jnp.broadcast_to(cmax - cmin, (L,))
        qs = jnp.full((L,), NQ, jnp.float32) / jnp.maximum(span, 1e-20)

        @plsc.parallel_loop(0, NQ * L, step=4 * L)
        def _(i):
            for u in range(4):
                qhist[pl.ds(i + u * L, L)] = z16i

        def quant(v):
            qf = jnp.minimum((v - cmin) * qs, jnp.float32(NQ - 1))
            qf = jnp.maximum(qf, jnp.float32(0.0))
            return qf.astype(jnp.int32)

        def qh(i, _, r=r):
            for u in range(2):
                v = cand[pl.ds(r * CAP + (i * 2 + u) * L, L)]
                plsc.addupdate_scatter(qhist, [quant(v) * L + lane], ones_i)
            return ()
        lax.fori_loop(0, CAP // L // 2, qh, ())

        def sc(g, base):
            for u in range(2):
                v = qhist[pl.ds((g * 2 + u) * L, L)]
                excl = plsc.cumsum(v) - v + base
                qhist[pl.ds((g * 2 + u) * L, L)] = excl
                base = base + jnp.sum(v)
            return base
        lax.fori_loop(0, NQ // 2, sc, jnp.int32(0))

        def st(i, _, r=r):
            v = cand[pl.ds(r * CAP + i * L, L)]
            qi = quant(v) * L + lane
            pos = plsc.load_gather(qhist, [qi])
            plsc.store_scatter(dst, [pos], v)
            plsc.addupdate_scatter(qhist, [qi], ones_i)
            return ()
        lax.fori_loop(0, CAP // L, st, ())

        @plsc.parallel_loop(0, VALIDSIZE, step=5 * L)
        def _(i, r=r, tv=tv):
            for u in range(5):
                v = dst[pl.ds(i + u * L, L)]
                outb[r, pl.ds(i + u * L, L)] = jnp.where(v <= tv, v, 0.0)

    pltpu.sync_copy(outb, out_hbm.at[pl.ds(c8, RW), :])


def _select_stage(dist, thd16):
    mesh = plsc.VectorSubcoreMesh(core_axis_name="c", subcore_axis_name="s")
    return pl.kernel(
        _select_body,
        out_type=jax.ShapeDtypeStruct((NCLUSTER, VALIDSIZE), jnp.float32),
        mesh=mesh,
        scratch_types=[
            pltpu.VMEM((RW, W), jnp.float32),
            pltpu.VMEM((RW, W), jnp.float32),
            pltpu.VMEM((RW * L,), jnp.float32),
            pltpu.VMEM((RW * NB * NH * L,), jnp.int32),
            pltpu.VMEM((RW * CAP,), jnp.float32),
            pltpu.VMEM((NQ * L,), jnp.int32),
            pltpu.VMEM((CAP,), jnp.float32),
            pltpu.VMEM((RW, VALIDSIZE), jnp.float32),
            pltpu.SemaphoreType.DMA,
            pltpu.SemaphoreType.DMA,
        ],
        compiler_params=pltpu.CompilerParams(needs_layout_passes=False),
    )(dist, thd16)


def kernel(data, label, cluster, weight, profit):
    dataT = jnp.pad(data, ((0, NPAD - NPOINTS), (0, 0))).T
    labp = jnp.pad(label, (0, NPAD - NPOINTS)).reshape(1, NPAD)
    wc = weight * cluster
    wc2 = -2.0 * wc
    ksum = jnp.sum(wc * cluster, axis=1, keepdims=True)
    thd = profit[:, 3:4]
    dist, cnt, slab = _dist_stage(dataT, labp, weight, wc2, ksum, thd)
    thd16 = jnp.broadcast_to(thd, (NCLUSTER, L)).reshape(-1)
    sel_vals = _select_stage(dist, thd16)
    cnt1 = cnt[:, 0]
    prof = slab[:, 0] / jnp.maximum(cnt1, 1.0)
    stat = jnp.stack([prof, cnt1], axis=1)
    return sel_vals, stat
